# baseline (speedup 1.0000x reference)
import functools

import jax
import jax.numpy as jnp
from jax import lax
from jax.experimental import pallas as pl
from jax.experimental.pallas import tpu as pltpu
from jax.experimental.pallas import tpu_sc as plsc

WINDOW = 128
NUM_CORES = 2
NUM_SUBCORES = 16
NUM_WORKERS = NUM_CORES * NUM_SUBCORES

SPREAD_LANES = 4096


def _spread_body(wt_ref, out_ref):
    x = wt_ref[...]
    out_ref[:, 0:32] = jnp.swapaxes(x, 0, 1)


def _unpack_body(x_ref, o_ref):
    x = x_ref[...]
    y = jnp.swapaxes(x, 0, 1)
    o_ref[...] = y.reshape(o_ref.shape)


def kernel(weights, indices):
    vocab, embed_dim = weights.shape
    batch, hist_len = indices.shape
    num_idx = batch * hist_len
    n_win = num_idx // WINDOW
    wpw = n_win // NUM_WORKERS
    ipw = wpw * WINDOW

    flat_idx = indices.reshape(num_idx) * 4

    wt = weights.T
    n_spread = (vocab + SPREAD_LANES - 1) // SPREAD_LANES

    @functools.partial(
        pl.kernel,
        mesh=pltpu.create_tensorcore_mesh("core"),
        out_type=jax.ShapeDtypeStruct((vocab, 128), weights.dtype),
    )
    def spread_kernel(wt_hbm, out_hbm):
        def body(wt_vmem, out_vmem):
            out_vmem[:, 0:32] = jnp.swapaxes(wt_vmem[...], 0, 1)

        pltpu.emit_pipeline(
            body,
            grid=(n_spread,),
            in_specs=[pl.BlockSpec((embed_dim, SPREAD_LANES), lambda i: (0, i))],
            out_specs=[pl.BlockSpec((SPREAD_LANES, 128), lambda i: (i, 0))],
            core_axis_name="core",
            dimension_semantics=(pltpu.PARALLEL,),
        )(wt_hbm, out_hbm)

    w128 = spread_kernel(wt)
    w4 = w128.reshape(4 * vocab, embed_dim)

    mesh = plsc.VectorSubcoreMesh(core_axis_name="c", subcore_axis_name="s")

    @functools.partial(
        pl.kernel,
        mesh=mesh,
        compiler_params=pltpu.CompilerParams(use_tc_tiling_on_sc=False),
        out_type=jax.ShapeDtypeStruct((num_idx, embed_dim), weights.dtype),
        scratch_types=[
            pltpu.VMEM((ipw,), jnp.int32),
            pltpu.VMEM((WINDOW, embed_dim), jnp.float32),
            pltpu.SemaphoreType.DMA,
        ],
    )
    def gather_kernel(table_hbm, idx_hbm, out_hbm, idx_v, rows_v, sem):
        wid = lax.axis_index("s") * NUM_CORES + lax.axis_index("c")
        base = wid * ipw
        pltpu.sync_copy(idx_hbm.at[pl.ds(base, ipw)], idx_v)

        @pl.loop(0, wpw)
        def _(j):
            pltpu.async_copy(
                table_hbm.at[idx_v.at[pl.ds(j * WINDOW, WINDOW)]], rows_v, sem
            ).wait()
            pltpu.sync_copy(rows_v, out_hbm.at[pl.ds(base + j * WINDOW, WINDOW)])

    out = gather_kernel(w4, flat_idx)

    row = hist_len * embed_dim
    xb = out.reshape(batch, row)

    @functools.partial(
        pl.kernel,
        mesh=pltpu.create_tensorcore_mesh("core"),
        out_type=jax.ShapeDtypeStruct(
            (hist_len, embed_dim, batch), weights.dtype
        ),
    )
    def unpack_kernel(x_hbm, o_hbm):
        def body(x_vmem, o_vmem):
            y = jnp.swapaxes(x_vmem[...], 0, 1)
            o_vmem[...] = y.reshape(o_vmem.shape)

        pltpu.emit_pipeline(
            body,
            grid=(batch // 128,),
            in_specs=[pl.BlockSpec((128, row), lambda i: (i, 0))],
            out_specs=[
                pl.BlockSpec((hist_len, embed_dim, 128), lambda i: (0, 0, i))
            ],
            core_axis_name="core",
            dimension_semantics=(pltpu.PARALLEL,),
        )(x_hbm, o_hbm)

    ot = unpack_kernel(xb)
    return jnp.transpose(ot, (2, 0, 1))
